# Initial kernel scaffold; baseline (speedup 1.0000x reference)
import jax
import jax.numpy as jnp
from jax.experimental import pallas as pl


def kernel(x, edge_index, batch, W1, as1, ad1, b1, W2, as2, ad2, b2, W3, as3, ad3, b3, W4, as4, ad4, b4, fcW, fcb):
    raise NotImplementedError("write your pallas kernel here")


# speedup vs baseline: 37.1835x; 37.1835x over previous
import functools

import jax
import jax.numpy as jnp
import numpy as np
from jax import lax
from jax.experimental import pallas as pl
from jax.experimental.pallas import tpu as pltpu
from jax.experimental.pallas import tpu_sc as plsc

N = 10000
E = 320000
G = 64
NC, NS, LANES = 2, 16, 16

N1 = 10240
ROWS_PER_TILE = N1 // NS
EPAD = 331776
EPT = EPAD // NS
EB = 64
RB = 2560


def _vperm(v, idx):
    dn = lax.GatherDimensionNumbers(offset_dims=(), collapsed_slice_dims=(0,),
                                    start_index_map=(0,))
    return lax.gather(v, idx[:, None], dn, (1,),
                      mode=lax.GatherScatterMode.PROMISE_IN_BOUNDS)


def _sc_layer_body(h_hbm, als_d_hbm, src_hbm, dst_hbm, out_hbm,
                   acc_sp, hbuf, hbuf2, dbuf, srcv, dstv, dstv2,
                   *, wc, roww, c_per_head):
    c = lax.axis_index("c")
    s = lax.axis_index("s")
    r0 = s * ROWS_PER_TILE
    base = c * N1

    z16 = jnp.zeros((16,), jnp.float32)
    for j in range(EB):
        for k in range(roww // 16):
            hbuf[j, pl.ds(k * 16, 16)] = z16
    for q in range(ROWS_PER_TILE // EB):
        pltpu.sync_copy(hbuf, acc_sp.at[pl.ds(r0 + q * EB, EB)])
    plsc.subcore_barrier()

    iota = lax.iota(jnp.int32, 16)
    qrow = iota // 4
    qcol = iota - qrow * 4
    pats = []
    for k in range(roww // 16):
        ch = iota + k * 16
        pats.append(jnp.where(ch < wc, ch // c_per_head,
                              jnp.where(ch < wc + 4, ch - wc, 0)))
    e0 = s * EPT

    def blk(b, carry):
        for half, hb in ((0, hbuf), (1, hbuf2)):
            off = e0 + (2 * b + half) * EB
            pltpu.sync_copy(src_hbm.at[pl.ds(off, EB)], srcv)
            pltpu.sync_copy(dst_hbm.at[pl.ds(off, EB)], dstv)
            for g in range(EB // 16):
                sl = pl.ds(g * 16, 16)
                srcv[sl] = srcv[sl] + base
                dstv2[sl] = dstv[sl] + base
            pltpu.sync_copy(h_hbm.at[srcv], hb)
            pltpu.sync_copy(als_d_hbm.at[dstv2], dbuf)
            for g in range(EB // 4):
                row = qrow + g * 4
                av = plsc.load_gather(hb, [row, qcol + (wc + 4)])
                dv = plsc.load_gather(dbuf, [row, qcol])
                e = av + dv
                e = jnp.where(e >= 0.0, e, 0.2 * e)
                ee_vec = jnp.exp(e)
                for jl in range(4):
                    j = g * 4 + jl
                    for k in range(roww // 16):
                        bv = _vperm(ee_vec, pats[k] + (jl * 4))
                        hv = hb[j, pl.ds(k * 16, 16)]
                        hb[j, pl.ds(k * 16, 16)] = hv * bv
            pltpu.sync_copy(hb, acc_sp.at[dstv], add=True)
        return carry

    lax.fori_loop(0, EPT // (2 * EB), blk, 0)
    plsc.subcore_barrier()
    for q in range(ROWS_PER_TILE // EB):
        rr = r0 + q * EB
        pltpu.sync_copy(acc_sp.at[pl.ds(rr, EB)], hbuf)
        pltpu.sync_copy(hbuf, out_hbm.at[c, pl.ds(rr, EB)])


def _sc_layer(h_ext, als_d, src, dst, wc, roww, c_per_head):
    mesh = plsc.VectorSubcoreMesh(core_axis_name="c", subcore_axis_name="s")
    body = functools.partial(_sc_layer_body, wc=wc, roww=roww,
                             c_per_head=c_per_head)
    return pl.kernel(
        body,
        out_type=jax.ShapeDtypeStruct((NC, N1, roww), jnp.float32),
        mesh=mesh,
        compiler_params=pltpu.CompilerParams(needs_layout_passes=False,
                                             use_tc_tiling_on_sc=False),
        scratch_types=[
            pltpu.VMEM_SHARED((N1, roww), jnp.float32),
            pltpu.VMEM((EB, roww), jnp.float32),
            pltpu.VMEM((EB, roww), jnp.float32),
            pltpu.VMEM((EB, 16), jnp.float32),
            pltpu.VMEM((EB,), jnp.int32),
            pltpu.VMEM((EB,), jnp.int32),
            pltpu.VMEM((EB,), jnp.int32),
        ],
    )(h_ext, als_d, src, dst)


def _tc_first_body(x_ref, w_ref, a_ref, h_ref, als_ref):
    h = jnp.dot(x_ref[...], w_ref[...], preferred_element_type=jnp.float32)
    h_ref[...] = h
    als_ref[...] = jnp.dot(h, a_ref[...], preferred_element_type=jnp.float32)


def _tc_layer_body(acc_ref, den_ref, b_ref, w_ref, a_ref, h_ref, als_ref):
    den = den_ref[...]
    t = acc_ref[...] / den + b_ref[...]
    x = jnp.where(den > 0.0, jnp.where(t > 0.0, t, jnp.exp(t) - 1.0), 0.0)
    h = jnp.dot(x, w_ref[...], preferred_element_type=jnp.float32)
    h_ref[...] = h
    als_ref[...] = jnp.dot(h, a_ref[...], preferred_element_type=jnp.float32)


def _tc_first(x, w, a_all):
    hc = w.shape[1]
    return pl.pallas_call(
        _tc_first_body,
        grid=(N1 // RB,),
        in_specs=[
            pl.BlockSpec((RB, x.shape[1]), lambda i: (i, 0)),
            pl.BlockSpec(w.shape, lambda i: (0, 0)),
            pl.BlockSpec(a_all.shape, lambda i: (0, 0)),
        ],
        out_specs=[
            pl.BlockSpec((RB, hc), lambda i: (i, 0)),
            pl.BlockSpec((RB, 16), lambda i: (i, 0)),
        ],
        out_shape=[
            jax.ShapeDtypeStruct((N1, hc), jnp.float32),
            jax.ShapeDtypeStruct((N1, 16), jnp.float32),
        ],
    )(x, w, a_all)


def _tc_layer(acc, den_exp, b, w, a_all):
    hc_in = acc.shape[1]
    hc = w.shape[1]
    return pl.pallas_call(
        _tc_layer_body,
        grid=(N1 // RB,),
        in_specs=[
            pl.BlockSpec((RB, hc_in), lambda i: (i, 0)),
            pl.BlockSpec((RB, hc_in), lambda i: (i, 0)),
            pl.BlockSpec((1, hc_in), lambda i: (0, 0)),
            pl.BlockSpec(w.shape, lambda i: (0, 0)),
            pl.BlockSpec(a_all.shape, lambda i: (0, 0)),
        ],
        out_specs=[
            pl.BlockSpec((RB, hc), lambda i: (i, 0)),
            pl.BlockSpec((RB, 16), lambda i: (i, 0)),
        ],
        out_shape=[
            jax.ShapeDtypeStruct((N1, hc), jnp.float32),
            jax.ShapeDtypeStruct((N1, 16), jnp.float32),
        ],
    )(acc, den_exp, b, w, a_all)


def _pool_body(acc_ref, den_ref, b_ref, batch_ref, sum_ref, max_ref, cnt_ref):
    i = pl.program_id(0)

    @pl.when(i == 0)
    def _init():
        sum_ref[...] = jnp.zeros_like(sum_ref)
        cnt_ref[...] = jnp.zeros_like(cnt_ref)
        max_ref[...] = jnp.full_like(max_ref, -jnp.inf)

    den = den_ref[...]
    t = acc_ref[...] / den + b_ref[...]
    x = jnp.where(den > 0.0, jnp.where(t > 0.0, t, jnp.exp(t) - 1.0), 0.0)

    bt = batch_ref[...]
    gids = lax.broadcasted_iota(jnp.int32, (1, G), 1)
    oh = (bt == gids).astype(jnp.float32)
    sum_ref[...] += lax.dot_general(oh, x, (((0,), (0,)), ((), ())),
                                    preferred_element_type=jnp.float32)
    cnt_ref[...] += lax.dot_general(oh, jnp.ones_like(x),
                                    (((0,), (0,)), ((), ())),
                                    preferred_element_type=jnp.float32)
    for g in range(G):
        row = jnp.max(jnp.where(bt == g, x, -jnp.inf), axis=0, keepdims=True)
        max_ref[g:g + 1, :] = jnp.maximum(max_ref[g:g + 1, :], row)


def _pool(acc, den_exp, b, batch2d):
    return pl.pallas_call(
        _pool_body,
        grid=(N1 // RB,),
        in_specs=[
            pl.BlockSpec((RB, 128), lambda i: (i, 0)),
            pl.BlockSpec((RB, 128), lambda i: (i, 0)),
            pl.BlockSpec((1, 128), lambda i: (0, 0)),
            pl.BlockSpec((RB, 1), lambda i: (i, 0)),
        ],
        out_specs=[
            pl.BlockSpec((G, 128), lambda i: (0, 0)),
            pl.BlockSpec((G, 128), lambda i: (0, 0)),
            pl.BlockSpec((G, 128), lambda i: (0, 0)),
        ],
        out_shape=[
            jax.ShapeDtypeStruct((G, 128), jnp.float32),
            jax.ShapeDtypeStruct((G, 128), jnp.float32),
            jax.ShapeDtypeStruct((G, 128), jnp.float32),
        ],
    )(acc, den_exp, b, batch2d)


def _head_body(sum_ref, max_ref, cnt_ref, w1_ref, w2_ref, fb_ref, out_ref):
    cnt = cnt_ref[...]
    mean = sum_ref[...] / jnp.maximum(cnt, 1.0)
    mx = jnp.where(cnt > 0.0, max_ref[...], 0.0)
    logits = (jnp.dot(mean, w1_ref[...], preferred_element_type=jnp.float32)
              + jnp.dot(mx, w2_ref[...], preferred_element_type=jnp.float32)
              + fb_ref[...])
    m = jnp.max(logits, axis=1, keepdims=True)
    lse = m + jnp.log(jnp.sum(jnp.exp(logits - m), axis=1, keepdims=True))
    out_ref[...] = logits - lse


def _head(sums, mx, cnt, fw1, fw2, fb):
    return pl.pallas_call(
        _head_body,
        out_shape=jax.ShapeDtypeStruct((G, 6), jnp.float32),
    )(sums, mx, cnt, fw1, fw2, fb)


def _block_diag(a):
    h, c = a.shape
    return (jnp.einsum('kc,hk->hck', a, jnp.eye(h, dtype=a.dtype))
            .reshape(h * c, h))


def _split_tables(h, als, wc, roww):
    ones = jnp.ones((N1, 4), jnp.float32)
    zpad = jnp.zeros((N1, roww - wc - 8), jnp.float32)
    h_ext = jnp.concatenate([
        jnp.concatenate([h[:, :wc], ones, als[:, 0:4], zpad], axis=1),
        jnp.concatenate([h[:, wc:], ones, als[:, 4:8], zpad], axis=1),
    ], axis=0)
    als_d = jnp.concatenate([als[:, 8:12], als[:, 12:16]], axis=0)
    als_d = jnp.concatenate(
        [als_d, jnp.zeros((NC * N1, 12), jnp.float32)], axis=1)
    return h_ext, als_d


def _merge(out, wc, c_per_head):
    acc = jnp.concatenate([out[0, :, :wc], out[1, :, :wc]], axis=1)
    den8 = jnp.concatenate([out[0, :, wc:wc + 4], out[1, :, wc:wc + 4]],
                           axis=1)
    den_exp = jnp.repeat(den8, c_per_head, axis=1)
    return acc, den_exp


def kernel(x, edge_index, batch, W1, as1, ad1, b1, W2, as2, ad2, b2,
           W3, as3, ad3, b3, W4, as4, ad4, b4, fcW, fcb):
    xp = jnp.concatenate(
        [x, jnp.zeros((N1 - N, x.shape[1]), jnp.float32)], axis=0)
    loop = jnp.arange(N, dtype=jnp.int32)
    fill = jnp.full((EPAD - E - N,), N, jnp.int32)
    src = jnp.concatenate([edge_index[0].astype(jnp.int32), loop, fill])
    dst = jnp.concatenate([edge_index[1].astype(jnp.int32), loop, fill])
    batch_p = jnp.concatenate(
        [batch.astype(jnp.int32), jnp.full((N1 - N,), 2 ** 24, jnp.int32)]
    ).reshape(N1, 1)

    layers = [
        (W1, as1, ad1, b1, 8, 32, 48),
        (W2, as2, ad2, b2, 16, 64, 80),
        (W3, as3, ad3, b3, 16, 64, 80),
        (W4, as4, ad4, b4, 16, 64, 80),
    ]

    acc = den_exp = None
    prev_b = None
    for li, (W, a_s, a_d, b, cph, wc, roww) in enumerate(layers):
        a_all = jnp.concatenate([_block_diag(a_s), _block_diag(a_d)], axis=1)
        if li == 0:
            h, als = _tc_first(xp, W, a_all)
        else:
            h, als = _tc_layer(acc, den_exp, prev_b.reshape(1, -1), W, a_all)
        h_ext, als_d = _split_tables(h, als, wc, roww)
        out = _sc_layer(h_ext, als_d, src, dst, wc, roww, cph)
        acc, den_exp = _merge(out, wc, cph)
        prev_b = b

    sums, mx, cnt = _pool(acc, den_exp, b4.reshape(1, -1), batch_p)
    return _head(sums, mx, cnt, fcW[:128], fcW[128:], fcb.reshape(1, 6))


# speedup vs baseline: 47.3702x; 1.2740x over previous
import functools

import jax
import jax.numpy as jnp
import numpy as np
from jax import lax
from jax.experimental import pallas as pl
from jax.experimental.pallas import tpu as pltpu
from jax.experimental.pallas import tpu_sc as plsc

N = 10000
E = 320000
G = 64
NC, NS, LANES = 2, 16, 16

N1 = 10240
ROWS_PER_TILE = N1 // NS
EPAD = 331776
EPT = EPAD // NS
EB = 128
RB = 2560


def _vperm(v, idx):
    dn = lax.GatherDimensionNumbers(offset_dims=(), collapsed_slice_dims=(0,),
                                    start_index_map=(0,))
    return lax.gather(v, idx[:, None], dn, (1,),
                      mode=lax.GatherScatterMode.PROMISE_IN_BOUNDS)


def _sc_layer_body(h_hbm, als_d_hbm, src_hbm, dst_hbm, out_hbm,
                   acc_sp, hbuf, hbuf2, dbuf, srcv, dstv, dstv2,
                   *, wc, roww, c_per_head):
    c = lax.axis_index("c")
    s = lax.axis_index("s")
    r0 = s * ROWS_PER_TILE
    base = c * N1

    z16 = jnp.zeros((16,), jnp.float32)
    for j in range(EB):
        for k in range(roww // 16):
            hbuf[j, pl.ds(k * 16, 16)] = z16
    for q in range(ROWS_PER_TILE // EB):
        pltpu.sync_copy(hbuf, acc_sp.at[pl.ds(r0 + q * EB, EB)])
    plsc.subcore_barrier()

    iota = lax.iota(jnp.int32, 16)
    qrow = iota // 4
    qcol = iota - qrow * 4
    pats = []
    for k in range(roww // 16):
        ch = iota + k * 16
        pats.append(jnp.where(ch < wc, ch // c_per_head,
                              jnp.where(ch < wc + 4, ch - wc, 0)))
    e0 = s * EPT

    def blk(b, carry):
        for half, hb in ((0, hbuf), (1, hbuf2)):
            off = e0 + (2 * b + half) * EB
            pltpu.sync_copy(src_hbm.at[pl.ds(off, EB)], srcv)
            pltpu.sync_copy(dst_hbm.at[pl.ds(off, EB)], dstv)
            for g in range(EB // 16):
                sl = pl.ds(g * 16, 16)
                srcv[sl] = srcv[sl] + base
                dstv2[sl] = dstv[sl] + base
            pltpu.sync_copy(h_hbm.at[srcv], hb)
            pltpu.sync_copy(als_d_hbm.at[dstv2], dbuf)
            for g in range(EB // 4):
                row = qrow + g * 4
                av = plsc.load_gather(hb, [row, qcol + (wc + 4)])
                dv = plsc.load_gather(dbuf, [row, qcol])
                e = av + dv
                e = jnp.where(e >= 0.0, e, 0.2 * e)
                ee_vec = jnp.exp(e)
                for jl in range(4):
                    j = g * 4 + jl
                    for k in range(roww // 16):
                        bv = _vperm(ee_vec, pats[k] + (jl * 4))
                        hv = hb[j, pl.ds(k * 16, 16)]
                        hb[j, pl.ds(k * 16, 16)] = hv * bv
            pltpu.sync_copy(hb, acc_sp.at[dstv], add=True)
        return carry

    lax.fori_loop(0, EPT // (2 * EB), blk, 0)
    plsc.subcore_barrier()
    for q in range(ROWS_PER_TILE // EB):
        rr = r0 + q * EB
        pltpu.sync_copy(acc_sp.at[pl.ds(rr, EB)], hbuf)
        pltpu.sync_copy(hbuf, out_hbm.at[c, pl.ds(rr, EB)])


def _sc_layer(h_ext, als_d, src, dst, wc, roww, c_per_head):
    mesh = plsc.VectorSubcoreMesh(core_axis_name="c", subcore_axis_name="s")
    body = functools.partial(_sc_layer_body, wc=wc, roww=roww,
                             c_per_head=c_per_head)
    return pl.kernel(
        body,
        out_type=jax.ShapeDtypeStruct((NC, N1, roww), jnp.float32),
        mesh=mesh,
        compiler_params=pltpu.CompilerParams(needs_layout_passes=False,
                                             use_tc_tiling_on_sc=False),
        scratch_types=[
            pltpu.VMEM_SHARED((N1, roww), jnp.float32),
            pltpu.VMEM((EB, roww), jnp.float32),
            pltpu.VMEM((EB, roww), jnp.float32),
            pltpu.VMEM((EB, 16), jnp.float32),
            pltpu.VMEM((EB,), jnp.int32),
            pltpu.VMEM((EB,), jnp.int32),
            pltpu.VMEM((EB,), jnp.int32),
        ],
    )(h_ext, als_d, src, dst)


def _tc_first_body(x_ref, w_ref, a_ref, h_ref, als_ref):
    h = jnp.dot(x_ref[...], w_ref[...], preferred_element_type=jnp.float32)
    h_ref[...] = h
    als_ref[...] = jnp.dot(h, a_ref[...], preferred_element_type=jnp.float32)


def _tc_layer_body(acc_ref, den_ref, b_ref, w_ref, a_ref, h_ref, als_ref):
    den = den_ref[...]
    t = acc_ref[...] / den + b_ref[...]
    x = jnp.where(den > 0.0, jnp.where(t > 0.0, t, jnp.exp(t) - 1.0), 0.0)
    h = jnp.dot(x, w_ref[...], preferred_element_type=jnp.float32)
    h_ref[...] = h
    als_ref[...] = jnp.dot(h, a_ref[...], preferred_element_type=jnp.float32)


def _tc_first(x, w, a_all):
    hc = w.shape[1]
    return pl.pallas_call(
        _tc_first_body,
        grid=(N1 // RB,),
        in_specs=[
            pl.BlockSpec((RB, x.shape[1]), lambda i: (i, 0)),
            pl.BlockSpec(w.shape, lambda i: (0, 0)),
            pl.BlockSpec(a_all.shape, lambda i: (0, 0)),
        ],
        out_specs=[
            pl.BlockSpec((RB, hc), lambda i: (i, 0)),
            pl.BlockSpec((RB, 16), lambda i: (i, 0)),
        ],
        out_shape=[
            jax.ShapeDtypeStruct((N1, hc), jnp.float32),
            jax.ShapeDtypeStruct((N1, 16), jnp.float32),
        ],
    )(x, w, a_all)


def _tc_layer(acc, den_exp, b, w, a_all):
    hc_in = acc.shape[1]
    hc = w.shape[1]
    return pl.pallas_call(
        _tc_layer_body,
        grid=(N1 // RB,),
        in_specs=[
            pl.BlockSpec((RB, hc_in), lambda i: (i, 0)),
            pl.BlockSpec((RB, hc_in), lambda i: (i, 0)),
            pl.BlockSpec((1, hc_in), lambda i: (0, 0)),
            pl.BlockSpec(w.shape, lambda i: (0, 0)),
            pl.BlockSpec(a_all.shape, lambda i: (0, 0)),
        ],
        out_specs=[
            pl.BlockSpec((RB, hc), lambda i: (i, 0)),
            pl.BlockSpec((RB, 16), lambda i: (i, 0)),
        ],
        out_shape=[
            jax.ShapeDtypeStruct((N1, hc), jnp.float32),
            jax.ShapeDtypeStruct((N1, 16), jnp.float32),
        ],
    )(acc, den_exp, b, w, a_all)


def _pool_body(acc_ref, den_ref, b_ref, batch_ref, sum_ref, max_ref, cnt_ref):
    i = pl.program_id(0)

    @pl.when(i == 0)
    def _init():
        sum_ref[...] = jnp.zeros_like(sum_ref)
        cnt_ref[...] = jnp.zeros_like(cnt_ref)
        max_ref[...] = jnp.full_like(max_ref, -jnp.inf)

    den = den_ref[...]
    t = acc_ref[...] / den + b_ref[...]
    x = jnp.where(den > 0.0, jnp.where(t > 0.0, t, jnp.exp(t) - 1.0), 0.0)

    bt = batch_ref[...]
    gids = lax.broadcasted_iota(jnp.int32, (1, G), 1)
    oh = (bt == gids).astype(jnp.float32)
    sum_ref[...] += lax.dot_general(oh, x, (((0,), (0,)), ((), ())),
                                    preferred_element_type=jnp.float32)
    cnt_ref[...] += lax.dot_general(oh, jnp.ones_like(x),
                                    (((0,), (0,)), ((), ())),
                                    preferred_element_type=jnp.float32)
    for g in range(G):
        row = jnp.max(jnp.where(bt == g, x, -jnp.inf), axis=0, keepdims=True)
        max_ref[g:g + 1, :] = jnp.maximum(max_ref[g:g + 1, :], row)


def _pool(acc, den_exp, b, batch2d):
    return pl.pallas_call(
        _pool_body,
        grid=(N1 // RB,),
        in_specs=[
            pl.BlockSpec((RB, 128), lambda i: (i, 0)),
            pl.BlockSpec((RB, 128), lambda i: (i, 0)),
            pl.BlockSpec((1, 128), lambda i: (0, 0)),
            pl.BlockSpec((RB, 1), lambda i: (i, 0)),
        ],
        out_specs=[
            pl.BlockSpec((G, 128), lambda i: (0, 0)),
            pl.BlockSpec((G, 128), lambda i: (0, 0)),
            pl.BlockSpec((G, 128), lambda i: (0, 0)),
        ],
        out_shape=[
            jax.ShapeDtypeStruct((G, 128), jnp.float32),
            jax.ShapeDtypeStruct((G, 128), jnp.float32),
            jax.ShapeDtypeStruct((G, 128), jnp.float32),
        ],
    )(acc, den_exp, b, batch2d)


def _head_body(sum_ref, max_ref, cnt_ref, w1_ref, w2_ref, fb_ref, out_ref):
    cnt = cnt_ref[...]
    mean = sum_ref[...] / jnp.maximum(cnt, 1.0)
    mx = jnp.where(cnt > 0.0, max_ref[...], 0.0)
    logits = (jnp.dot(mean, w1_ref[...], preferred_element_type=jnp.float32)
              + jnp.dot(mx, w2_ref[...], preferred_element_type=jnp.float32)
              + fb_ref[...])
    m = jnp.max(logits, axis=1, keepdims=True)
    lse = m + jnp.log(jnp.sum(jnp.exp(logits - m), axis=1, keepdims=True))
    out_ref[...] = logits - lse


def _head(sums, mx, cnt, fw1, fw2, fb):
    return pl.pallas_call(
        _head_body,
        out_shape=jax.ShapeDtypeStruct((G, 6), jnp.float32),
    )(sums, mx, cnt, fw1, fw2, fb)


def _block_diag(a):
    h, c = a.shape
    return (jnp.einsum('kc,hk->hck', a, jnp.eye(h, dtype=a.dtype))
            .reshape(h * c, h))


def _split_tables(h, als, wc, roww):
    ones = jnp.ones((N1, 4), jnp.float32)
    zpad = jnp.zeros((N1, roww - wc - 8), jnp.float32)
    h_ext = jnp.concatenate([
        jnp.concatenate([h[:, :wc], ones, als[:, 0:4], zpad], axis=1),
        jnp.concatenate([h[:, wc:], ones, als[:, 4:8], zpad], axis=1),
    ], axis=0)
    als_d = jnp.concatenate([als[:, 8:12], als[:, 12:16]], axis=0)
    als_d = jnp.concatenate(
        [als_d, jnp.zeros((NC * N1, 12), jnp.float32)], axis=1)
    return h_ext, als_d


def _merge(out, wc, c_per_head):
    acc = jnp.concatenate([out[0, :, :wc], out[1, :, :wc]], axis=1)
    den8 = jnp.concatenate([out[0, :, wc:wc + 4], out[1, :, wc:wc + 4]],
                           axis=1)
    den_exp = jnp.repeat(den8, c_per_head, axis=1)
    return acc, den_exp


def kernel(x, edge_index, batch, W1, as1, ad1, b1, W2, as2, ad2, b2,
           W3, as3, ad3, b3, W4, as4, ad4, b4, fcW, fcb):
    xp = jnp.concatenate(
        [x, jnp.zeros((N1 - N, x.shape[1]), jnp.float32)], axis=0)
    loop = jnp.arange(N, dtype=jnp.int32)
    fill = jnp.full((EPAD - E - N,), N, jnp.int32)
    src = jnp.concatenate([edge_index[0].astype(jnp.int32), loop, fill])
    dst = jnp.concatenate([edge_index[1].astype(jnp.int32), loop, fill])
    batch_p = jnp.concatenate(
        [batch.astype(jnp.int32), jnp.full((N1 - N,), 2 ** 24, jnp.int32)]
    ).reshape(N1, 1)

    layers = [
        (W1, as1, ad1, b1, 8, 32, 48),
        (W2, as2, ad2, b2, 16, 64, 80),
        (W3, as3, ad3, b3, 16, 64, 80),
        (W4, as4, ad4, b4, 16, 64, 80),
    ]

    acc = den_exp = None
    prev_b = None
    for li, (W, a_s, a_d, b, cph, wc, roww) in enumerate(layers):
        a_all = jnp.concatenate([_block_diag(a_s), _block_diag(a_d)], axis=1)
        if li == 0:
            h, als = _tc_first(xp, W, a_all)
        else:
            h, als = _tc_layer(acc, den_exp, prev_b.reshape(1, -1), W, a_all)
        h_ext, als_d = _split_tables(h, als, wc, roww)
        out = _sc_layer(h_ext, als_d, src, dst, wc, roww, cph)
        acc, den_exp = _merge(out, wc, cph)
        prev_b = b

    sums, mx, cnt = _pool(acc, den_exp, b4.reshape(1, -1), batch_p)
    return _head(sums, mx, cnt, fcW[:128], fcW[128:], fcb.reshape(1, 6))


# speedup vs baseline: 57.4504x; 1.2128x over previous
import functools

import jax
import jax.numpy as jnp
import numpy as np
from jax import lax
from jax.experimental import pallas as pl
from jax.experimental.pallas import tpu as pltpu
from jax.experimental.pallas import tpu_sc as plsc

N = 10000
E = 320000
G = 64
NC, NS, LANES = 2, 16, 16

N1 = 10240
ROWS_PER_TILE = N1 // NS
EPAD = 331776
EPT = EPAD // NS
EB = 128
RB = 2560


def _vperm(v, idx):
    dn = lax.GatherDimensionNumbers(offset_dims=(), collapsed_slice_dims=(0,),
                                    start_index_map=(0,))
    return lax.gather(v, idx[:, None], dn, (1,),
                      mode=lax.GatherScatterMode.PROMISE_IN_BOUNDS)


def _sc_layer_body(h_hbm, als_d_hbm, src_hbm, dst_hbm, out_hbm,
                   acc_sp, hbuf, hbuf2, dbuf, srcv, dstv, dstv2, sem1, sem2,
                   *, wc, roww, c_per_head):
    c = lax.axis_index("c")
    s = lax.axis_index("s")
    r0 = s * ROWS_PER_TILE
    base = c * N1

    z16 = jnp.zeros((16,), jnp.float32)
    for j in range(EB):
        for k in range(roww // 16):
            hbuf[j, pl.ds(k * 16, 16)] = z16
    for q in range(ROWS_PER_TILE // EB):
        pltpu.sync_copy(hbuf, acc_sp.at[pl.ds(r0 + q * EB, EB)])
    plsc.subcore_barrier()

    iota = lax.iota(jnp.int32, 16)
    qrow = iota // 4
    qcol = iota - qrow * 4
    pats = []
    for k in range(roww // 16):
        ch = iota + k * 16
        pats.append(jnp.where(ch < wc, ch // c_per_head,
                              jnp.where(ch < wc + 4, ch - wc, 0)))
    e0 = s * EPT

    def blk(b, carry):
        for half, hb in ((0, hbuf), (1, hbuf2)):
            off = e0 + (2 * b + half) * EB
            c1 = pltpu.async_copy(src_hbm.at[pl.ds(off, EB)], srcv, sem1)
            c2 = pltpu.async_copy(dst_hbm.at[pl.ds(off, EB)], dstv, sem2)
            c1.wait()
            c2.wait()
            for g in range(EB // 16):
                sl = pl.ds(g * 16, 16)
                srcv[sl] = srcv[sl] + base
                dstv2[sl] = dstv[sl] + base
            g1 = pltpu.async_copy(h_hbm.at[srcv], hb, sem1)
            g2 = pltpu.async_copy(als_d_hbm.at[dstv2], dbuf, sem2)
            g1.wait()
            g2.wait()
            for g in range(EB // 4):
                row = qrow + g * 4
                av = plsc.load_gather(hb, [row, qcol + (wc + 4)])
                dv = plsc.load_gather(dbuf, [row, qcol])
                e = av + dv
                e = jnp.where(e >= 0.0, e, 0.2 * e)
                ee_vec = jnp.exp(e)
                for jl in range(4):
                    j = g * 4 + jl
                    for k in range(roww // 16):
                        bv = _vperm(ee_vec, pats[k] + (jl * 4))
                        hv = hb[j, pl.ds(k * 16, 16)]
                        hb[j, pl.ds(k * 16, 16)] = hv * bv
            pltpu.sync_copy(hb, acc_sp.at[dstv], add=True)
        return carry

    lax.fori_loop(0, EPT // (2 * EB), blk, 0)
    plsc.subcore_barrier()
    for q in range(ROWS_PER_TILE // EB):
        rr = r0 + q * EB
        pltpu.sync_copy(acc_sp.at[pl.ds(rr, EB)], hbuf)
        pltpu.sync_copy(hbuf, out_hbm.at[c, pl.ds(rr, EB)])


def _sc_layer(h_ext, als_d, src, dst, wc, roww, c_per_head):
    mesh = plsc.VectorSubcoreMesh(core_axis_name="c", subcore_axis_name="s")
    body = functools.partial(_sc_layer_body, wc=wc, roww=roww,
                             c_per_head=c_per_head)
    return pl.kernel(
        body,
        out_type=jax.ShapeDtypeStruct((NC, N1, roww), jnp.float32),
        mesh=mesh,
        compiler_params=pltpu.CompilerParams(needs_layout_passes=False,
                                             use_tc_tiling_on_sc=False),
        scratch_types=[
            pltpu.VMEM_SHARED((N1, roww), jnp.float32),
            pltpu.VMEM((EB, roww), jnp.float32),
            pltpu.VMEM((EB, roww), jnp.float32),
            pltpu.VMEM((EB, 16), jnp.float32),
            pltpu.VMEM((EB,), jnp.int32),
            pltpu.VMEM((EB,), jnp.int32),
            pltpu.VMEM((EB,), jnp.int32),
            pltpu.SemaphoreType.DMA,
            pltpu.SemaphoreType.DMA,
        ],
    )(h_ext, als_d, src, dst)


def _tc_first_body(x_ref, w_ref, a_ref, h_ref, als_ref):
    h = jnp.dot(x_ref[...], w_ref[...], preferred_element_type=jnp.float32)
    h_ref[...] = h
    als_ref[...] = jnp.dot(h, a_ref[...], preferred_element_type=jnp.float32)


def _tc_layer_body(acc_ref, den_ref, b_ref, w_ref, a_ref, h_ref, als_ref):
    den = den_ref[...]
    t = acc_ref[...] / den + b_ref[...]
    x = jnp.where(den > 0.0, jnp.where(t > 0.0, t, jnp.exp(t) - 1.0), 0.0)
    h = jnp.dot(x, w_ref[...], preferred_element_type=jnp.float32)
    h_ref[...] = h
    als_ref[...] = jnp.dot(h, a_ref[...], preferred_element_type=jnp.float32)


def _tc_first(x, w, a_all):
    hc = w.shape[1]
    return pl.pallas_call(
        _tc_first_body,
        grid=(N1 // RB,),
        in_specs=[
            pl.BlockSpec((RB, x.shape[1]), lambda i: (i, 0)),
            pl.BlockSpec(w.shape, lambda i: (0, 0)),
            pl.BlockSpec(a_all.shape, lambda i: (0, 0)),
        ],
        out_specs=[
            pl.BlockSpec((RB, hc), lambda i: (i, 0)),
            pl.BlockSpec((RB, 16), lambda i: (i, 0)),
        ],
        out_shape=[
            jax.ShapeDtypeStruct((N1, hc), jnp.float32),
            jax.ShapeDtypeStruct((N1, 16), jnp.float32),
        ],
    )(x, w, a_all)


def _tc_layer(acc, den_exp, b, w, a_all):
    hc_in = acc.shape[1]
    hc = w.shape[1]
    return pl.pallas_call(
        _tc_layer_body,
        grid=(N1 // RB,),
        in_specs=[
            pl.BlockSpec((RB, hc_in), lambda i: (i, 0)),
            pl.BlockSpec((RB, hc_in), lambda i: (i, 0)),
            pl.BlockSpec((1, hc_in), lambda i: (0, 0)),
            pl.BlockSpec(w.shape, lambda i: (0, 0)),
            pl.BlockSpec(a_all.shape, lambda i: (0, 0)),
        ],
        out_specs=[
            pl.BlockSpec((RB, hc), lambda i: (i, 0)),
            pl.BlockSpec((RB, 16), lambda i: (i, 0)),
        ],
        out_shape=[
            jax.ShapeDtypeStruct((N1, hc), jnp.float32),
            jax.ShapeDtypeStruct((N1, 16), jnp.float32),
        ],
    )(acc, den_exp, b, w, a_all)


def _pool_body(acc_ref, den_ref, b_ref, batch_ref, sum_ref, max_ref, cnt_ref):
    i = pl.program_id(0)

    @pl.when(i == 0)
    def _init():
        sum_ref[...] = jnp.zeros_like(sum_ref)
        cnt_ref[...] = jnp.zeros_like(cnt_ref)
        max_ref[...] = jnp.full_like(max_ref, -jnp.inf)

    den = den_ref[...]
    t = acc_ref[...] / den + b_ref[...]
    x = jnp.where(den > 0.0, jnp.where(t > 0.0, t, jnp.exp(t) - 1.0), 0.0)

    bt = batch_ref[...]
    gids = lax.broadcasted_iota(jnp.int32, (1, G), 1)
    oh = (bt == gids).astype(jnp.float32)
    sum_ref[...] += lax.dot_general(oh, x, (((0,), (0,)), ((), ())),
                                    preferred_element_type=jnp.float32)
    cnt_ref[...] += lax.dot_general(oh, jnp.ones_like(x),
                                    (((0,), (0,)), ((), ())),
                                    preferred_element_type=jnp.float32)
    for g in range(G):
        row = jnp.max(jnp.where(bt == g, x, -jnp.inf), axis=0, keepdims=True)
        max_ref[g:g + 1, :] = jnp.maximum(max_ref[g:g + 1, :], row)


def _pool(acc, den_exp, b, batch2d):
    return pl.pallas_call(
        _pool_body,
        grid=(N1 // RB,),
        in_specs=[
            pl.BlockSpec((RB, 128), lambda i: (i, 0)),
            pl.BlockSpec((RB, 128), lambda i: (i, 0)),
            pl.BlockSpec((1, 128), lambda i: (0, 0)),
            pl.BlockSpec((RB, 1), lambda i: (i, 0)),
        ],
        out_specs=[
            pl.BlockSpec((G, 128), lambda i: (0, 0)),
            pl.BlockSpec((G, 128), lambda i: (0, 0)),
            pl.BlockSpec((G, 128), lambda i: (0, 0)),
        ],
        out_shape=[
            jax.ShapeDtypeStruct((G, 128), jnp.float32),
            jax.ShapeDtypeStruct((G, 128), jnp.float32),
            jax.ShapeDtypeStruct((G, 128), jnp.float32),
        ],
    )(acc, den_exp, b, batch2d)


def _head_body(sum_ref, max_ref, cnt_ref, w1_ref, w2_ref, fb_ref, out_ref):
    cnt = cnt_ref[...]
    mean = sum_ref[...] / jnp.maximum(cnt, 1.0)
    mx = jnp.where(cnt > 0.0, max_ref[...], 0.0)
    logits = (jnp.dot(mean, w1_ref[...], preferred_element_type=jnp.float32)
              + jnp.dot(mx, w2_ref[...], preferred_element_type=jnp.float32)
              + fb_ref[...])
    m = jnp.max(logits, axis=1, keepdims=True)
    lse = m + jnp.log(jnp.sum(jnp.exp(logits - m), axis=1, keepdims=True))
    out_ref[...] = logits - lse


def _head(sums, mx, cnt, fw1, fw2, fb):
    return pl.pallas_call(
        _head_body,
        out_shape=jax.ShapeDtypeStruct((G, 6), jnp.float32),
    )(sums, mx, cnt, fw1, fw2, fb)


def _block_diag(a):
    h, c = a.shape
    return (jnp.einsum('kc,hk->hck', a, jnp.eye(h, dtype=a.dtype))
            .reshape(h * c, h))


def _split_tables(h, als, wc, roww):
    ones = jnp.ones((N1, 4), jnp.float32)
    zpad = jnp.zeros((N1, roww - wc - 8), jnp.float32)
    h_ext = jnp.concatenate([
        jnp.concatenate([h[:, :wc], ones, als[:, 0:4], zpad], axis=1),
        jnp.concatenate([h[:, wc:], ones, als[:, 4:8], zpad], axis=1),
    ], axis=0)
    als_d = jnp.concatenate([als[:, 8:12], als[:, 12:16]], axis=0)
    als_d = jnp.concatenate(
        [als_d, jnp.zeros((NC * N1, 12), jnp.float32)], axis=1)
    return h_ext, als_d


def _merge(out, wc, c_per_head):
    acc = jnp.concatenate([out[0, :, :wc], out[1, :, :wc]], axis=1)
    den8 = jnp.concatenate([out[0, :, wc:wc + 4], out[1, :, wc:wc + 4]],
                           axis=1)
    den_exp = jnp.repeat(den8, c_per_head, axis=1)
    return acc, den_exp


def kernel(x, edge_index, batch, W1, as1, ad1, b1, W2, as2, ad2, b2,
           W3, as3, ad3, b3, W4, as4, ad4, b4, fcW, fcb):
    xp = jnp.concatenate(
        [x, jnp.zeros((N1 - N, x.shape[1]), jnp.float32)], axis=0)
    loop = jnp.arange(N, dtype=jnp.int32)
    fill = jnp.full((EPAD - E - N,), N, jnp.int32)
    src = jnp.concatenate([edge_index[0].astype(jnp.int32), loop, fill])
    dst = jnp.concatenate([edge_index[1].astype(jnp.int32), loop, fill])
    batch_p = jnp.concatenate(
        [batch.astype(jnp.int32), jnp.full((N1 - N,), 2 ** 24, jnp.int32)]
    ).reshape(N1, 1)

    layers = [
        (W1, as1, ad1, b1, 8, 32, 48),
        (W2, as2, ad2, b2, 16, 64, 80),
        (W3, as3, ad3, b3, 16, 64, 80),
        (W4, as4, ad4, b4, 16, 64, 80),
    ]

    acc = den_exp = None
    prev_b = None
    for li, (W, a_s, a_d, b, cph, wc, roww) in enumerate(layers):
        a_all = jnp.concatenate([_block_diag(a_s), _block_diag(a_d)], axis=1)
        if li == 0:
            h, als = _tc_first(xp, W, a_all)
        else:
            h, als = _tc_layer(acc, den_exp, prev_b.reshape(1, -1), W, a_all)
        h_ext, als_d = _split_tables(h, als, wc, roww)
        out = _sc_layer(h_ext, als_d, src, dst, wc, roww, cph)
        acc, den_exp = _merge(out, wc, cph)
        prev_b = b

    sums, mx, cnt = _pool(acc, den_exp, b4.reshape(1, -1), batch_p)
    return _head(sums, mx, cnt, fcW[:128], fcW[128:], fcb.reshape(1, 6))


# speedup vs baseline: 57.4765x; 1.0005x over previous
import functools

import jax
import jax.numpy as jnp
from jax import lax
from jax.experimental import pallas as pl
from jax.experimental.pallas import tpu as pltpu
from jax.experimental.pallas import tpu_sc as plsc

N = 10000
E = 320000
G = 64
NC, NS, LANES = 2, 16, 16

N1 = 10240
ROWS_PER_TILE = N1 // NS
EPAD = 331776
EPT = EPAD // NS
EB = 128
RB = 2560


def _vperm(v, idx):
    dn = lax.GatherDimensionNumbers(offset_dims=(), collapsed_slice_dims=(0,),
                                    start_index_map=(0,))
    return lax.gather(v, idx[:, None], dn, (1,),
                      mode=lax.GatherScatterMode.PROMISE_IN_BOUNDS)


def _sc_layer_body(h_hbm, als_d_hbm, src_hbm, dst_hbm, out_hbm,
                   acc_sp, hbuf, hbuf2, dbuf, srcv, dstv, dstv2, sem1, sem2,
                   *, wc, roww, c_per_head):
    c = lax.axis_index("c")
    s = lax.axis_index("s")
    r0 = s * ROWS_PER_TILE
    base = c * N1

    z16 = jnp.zeros((16,), jnp.float32)
    for j in range(EB):
        for k in range(roww // 16):
            hbuf[j, pl.ds(k * 16, 16)] = z16
    for q in range(ROWS_PER_TILE // EB):
        pltpu.sync_copy(hbuf, acc_sp.at[pl.ds(r0 + q * EB, EB)])
    plsc.subcore_barrier()

    iota = lax.iota(jnp.int32, 16)
    qrow = iota // 4
    qcol = iota - qrow * 4
    pats = []
    for k in range(roww // 16):
        ch = iota + k * 16
        pats.append(jnp.where(ch < wc, ch // c_per_head,
                              jnp.where(ch < wc + 4, ch - wc, 0)))
    e0 = s * EPT

    def blk(b, carry):
        for half, hb in ((0, hbuf), (1, hbuf2)):
            off = e0 + (2 * b + half) * EB
            c1 = pltpu.async_copy(src_hbm.at[pl.ds(off, EB)], srcv, sem1)
            c2 = pltpu.async_copy(dst_hbm.at[pl.ds(off, EB)], dstv, sem2)
            c1.wait()
            c2.wait()
            for g in range(EB // 16):
                sl = pl.ds(g * 16, 16)
                srcv[sl] = srcv[sl] + base
                dstv2[sl] = dstv[sl] + base
            g1 = pltpu.async_copy(h_hbm.at[srcv], hb, sem1)
            g2 = pltpu.async_copy(als_d_hbm.at[dstv2], dbuf, sem2)
            g1.wait()
            g2.wait()
            for g in range(EB // 4):
                row = qrow + g * 4
                av = plsc.load_gather(hb, [row, qcol + (wc + 4)])
                dv = plsc.load_gather(dbuf, [row, qcol])
                e = av + dv
                e = jnp.where(e >= 0.0, e, 0.2 * e)
                ee_vec = jnp.exp(e)
                for jl in range(4):
                    j = g * 4 + jl
                    for k in range(roww // 16):
                        bv = _vperm(ee_vec, pats[k] + (jl * 4))
                        hv = hb[j, pl.ds(k * 16, 16)]
                        hb[j, pl.ds(k * 16, 16)] = hv * bv
            pltpu.sync_copy(hb, acc_sp.at[dstv], add=True)
        return carry

    lax.fori_loop(0, EPT // (2 * EB), blk, 0)
    plsc.subcore_barrier()
    for q in range(ROWS_PER_TILE // EB):
        rr = r0 + q * EB
        pltpu.sync_copy(acc_sp.at[pl.ds(rr, EB)], hbuf)
        pltpu.sync_copy(hbuf, out_hbm.at[c, pl.ds(rr, EB)])


def _sc_layer(h_ext, als_d, src, dst, wc, roww, c_per_head):
    mesh = plsc.VectorSubcoreMesh(core_axis_name="c", subcore_axis_name="s")
    body = functools.partial(_sc_layer_body, wc=wc, roww=roww,
                             c_per_head=c_per_head)
    return pl.kernel(
        body,
        out_type=jax.ShapeDtypeStruct((NC, N1, roww), jnp.float32),
        mesh=mesh,
        compiler_params=pltpu.CompilerParams(needs_layout_passes=False,
                                             use_tc_tiling_on_sc=False),
        scratch_types=[
            pltpu.VMEM_SHARED((N1, roww), jnp.float32),
            pltpu.VMEM((EB, roww), jnp.float32),
            pltpu.VMEM((EB, roww), jnp.float32),
            pltpu.VMEM((EB, 16), jnp.float32),
            pltpu.VMEM((EB,), jnp.int32),
            pltpu.VMEM((EB,), jnp.int32),
            pltpu.VMEM((EB,), jnp.int32),
            pltpu.SemaphoreType.DMA,
            pltpu.SemaphoreType.DMA,
        ],
    )(h_ext, als_d, src, dst)


def _tc_first_body(x_ref, w_ref, a_ref, h_ref, als_ref):
    h = jnp.dot(x_ref[...], w_ref[...], preferred_element_type=jnp.float32)
    h_ref[...] = h
    als_ref[...] = jnp.dot(h, a_ref[...], preferred_element_type=jnp.float32)


def _tc_layer_body(acc_ref, den_ref, b_ref, w_ref, a_ref, h_ref, als_ref):
    den = den_ref[...]
    t = acc_ref[...] / den + b_ref[...]
    x = jnp.where(den > 0.0, jnp.where(t > 0.0, t, jnp.exp(t) - 1.0), 0.0)
    h = jnp.dot(x, w_ref[...], preferred_element_type=jnp.float32)
    h_ref[...] = h
    als_ref[...] = jnp.dot(h, a_ref[...], preferred_element_type=jnp.float32)


def _tc_first(x, w, a_all):
    hc = w.shape[1]
    return pl.pallas_call(
        _tc_first_body,
        grid=(N1 // RB,),
        in_specs=[
            pl.BlockSpec((RB, x.shape[1]), lambda i: (i, 0)),
            pl.BlockSpec(w.shape, lambda i: (0, 0)),
            pl.BlockSpec(a_all.shape, lambda i: (0, 0)),
        ],
        out_specs=[
            pl.BlockSpec((RB, hc), lambda i: (i, 0)),
            pl.BlockSpec((RB, 16), lambda i: (i, 0)),
        ],
        out_shape=[
            jax.ShapeDtypeStruct((N1, hc), jnp.float32),
            jax.ShapeDtypeStruct((N1, 16), jnp.float32),
        ],
    )(x, w, a_all)


def _tc_layer(acc, den_exp, b, w, a_all):
    hc_in = acc.shape[1]
    hc = w.shape[1]
    return pl.pallas_call(
        _tc_layer_body,
        grid=(N1 // RB,),
        in_specs=[
            pl.BlockSpec((RB, hc_in), lambda i: (i, 0)),
            pl.BlockSpec((RB, hc_in), lambda i: (i, 0)),
            pl.BlockSpec((1, hc_in), lambda i: (0, 0)),
            pl.BlockSpec(w.shape, lambda i: (0, 0)),
            pl.BlockSpec(a_all.shape, lambda i: (0, 0)),
        ],
        out_specs=[
            pl.BlockSpec((RB, hc), lambda i: (i, 0)),
            pl.BlockSpec((RB, 16), lambda i: (i, 0)),
        ],
        out_shape=[
            jax.ShapeDtypeStruct((N1, hc), jnp.float32),
            jax.ShapeDtypeStruct((N1, 16), jnp.float32),
        ],
    )(acc, den_exp, b, w, a_all)


def _pool_body(acc_ref, den_ref, b_ref, batch_ref, sum_ref, max_ref, cnt_ref):
    i = pl.program_id(0)

    @pl.when(i == 0)
    def _init():
        sum_ref[...] = jnp.zeros_like(sum_ref)
        cnt_ref[...] = jnp.zeros_like(cnt_ref)
        max_ref[...] = jnp.full_like(max_ref, -jnp.inf)

    den = den_ref[...]
    t = acc_ref[...] / den + b_ref[...]
    x = jnp.where(den > 0.0, jnp.where(t > 0.0, t, jnp.exp(t) - 1.0), 0.0)

    bt = batch_ref[...]
    gids = lax.broadcasted_iota(jnp.int32, (1, G), 1)
    oh = (bt == gids).astype(jnp.float32)
    sum_ref[...] += lax.dot_general(oh, x, (((0,), (0,)), ((), ())),
                                    preferred_element_type=jnp.float32)
    cnt_ref[...] += lax.dot_general(oh, jnp.ones_like(x),
                                    (((0,), (0,)), ((), ())),
                                    preferred_element_type=jnp.float32)
    for g in range(G):
        row = jnp.max(jnp.where(bt == g, x, -jnp.inf), axis=0, keepdims=True)
        max_ref[g:g + 1, :] = jnp.maximum(max_ref[g:g + 1, :], row)


def _pool(acc, den_exp, b, batch2d):
    return pl.pallas_call(
        _pool_body,
        grid=(N1 // RB,),
        in_specs=[
            pl.BlockSpec((RB, 128), lambda i: (i, 0)),
            pl.BlockSpec((RB, 128), lambda i: (i, 0)),
            pl.BlockSpec((1, 128), lambda i: (0, 0)),
            pl.BlockSpec((RB, 1), lambda i: (i, 0)),
        ],
        out_specs=[
            pl.BlockSpec((G, 128), lambda i: (0, 0)),
            pl.BlockSpec((G, 128), lambda i: (0, 0)),
            pl.BlockSpec((G, 128), lambda i: (0, 0)),
        ],
        out_shape=[
            jax.ShapeDtypeStruct((G, 128), jnp.float32),
            jax.ShapeDtypeStruct((G, 128), jnp.float32),
            jax.ShapeDtypeStruct((G, 128), jnp.float32),
        ],
    )(acc, den_exp, b, batch2d)


def _head_body(sum_ref, max_ref, cnt_ref, w1_ref, w2_ref, fb_ref, out_ref):
    cnt = cnt_ref[...]
    mean = sum_ref[...] / jnp.maximum(cnt, 1.0)
    mx = jnp.where(cnt > 0.0, max_ref[...], 0.0)
    logits = (jnp.dot(mean, w1_ref[...], preferred_element_type=jnp.float32)
              + jnp.dot(mx, w2_ref[...], preferred_element_type=jnp.float32)
              + fb_ref[...])
    m = jnp.max(logits, axis=1, keepdims=True)
    lse = m + jnp.log(jnp.sum(jnp.exp(logits - m), axis=1, keepdims=True))
    out_ref[...] = logits - lse


def _head(sums, mx, cnt, fw1, fw2, fb):
    return pl.pallas_call(
        _head_body,
        out_shape=jax.ShapeDtypeStruct((G, 6), jnp.float32),
    )(sums, mx, cnt, fw1, fw2, fb)


def _block_diag(a):
    h, c = a.shape
    return (jnp.einsum('kc,hk->hck', a, jnp.eye(h, dtype=a.dtype))
            .reshape(h * c, h))


def _split_tables(h, als, wc, roww):
    ones = jnp.ones((N1, 4), jnp.float32)
    zpad = jnp.zeros((N1, roww - wc - 8), jnp.float32)
    h_ext = jnp.concatenate([
        jnp.concatenate([h[:, :wc], ones, als[:, 0:4], zpad], axis=1),
        jnp.concatenate([h[:, wc:], ones, als[:, 4:8], zpad], axis=1),
    ], axis=0)
    als_d = jnp.concatenate([als[:, 8:12], als[:, 12:16]], axis=0)
    als_d = jnp.concatenate(
        [als_d, jnp.zeros((NC * N1, 12), jnp.float32)], axis=1)
    return h_ext, als_d


def _merge(out, wc, c_per_head):
    acc = jnp.concatenate([out[0, :, :wc], out[1, :, :wc]], axis=1)
    den8 = jnp.concatenate([out[0, :, wc:wc + 4], out[1, :, wc:wc + 4]],
                           axis=1)
    den_exp = jnp.repeat(den8, c_per_head, axis=1)
    return acc, den_exp


def kernel(x, edge_index, batch, W1, as1, ad1, b1, W2, as2, ad2, b2,
           W3, as3, ad3, b3, W4, as4, ad4, b4, fcW, fcb):
    xp = jnp.concatenate(
        [x, jnp.zeros((N1 - N, x.shape[1]), jnp.float32)], axis=0)
    loop = jnp.arange(N, dtype=jnp.int32)
    fill = jnp.full((EPAD - E - N,), N, jnp.int32)
    src = jnp.concatenate([edge_index[0].astype(jnp.int32), loop, fill])
    dst = jnp.concatenate([edge_index[1].astype(jnp.int32), loop, fill])
    batch_p = jnp.concatenate(
        [batch.astype(jnp.int32), jnp.full((N1 - N,), 2 ** 24, jnp.int32)]
    ).reshape(N1, 1)

    layers = [
        (W1, as1, ad1, b1, 8, 32, 48),
        (W2, as2, ad2, b2, 16, 64, 80),
        (W3, as3, ad3, b3, 16, 64, 80),
        (W4, as4, ad4, b4, 16, 64, 80),
    ]

    acc = den_exp = None
    prev_b = None
    for li, (W, a_s, a_d, b, cph, wc, roww) in enumerate(layers):
        a_all = jnp.concatenate([_block_diag(a_s), _block_diag(a_d)], axis=1)
        if li == 0:
            h, als = _tc_first(xp, W, a_all)
        else:
            h, als = _tc_layer(acc, den_exp, prev_b.reshape(1, -1), W, a_all)
        h_ext, als_d = _split_tables(h, als, wc, roww)
        out = _sc_layer(h_ext, als_d, src, dst, wc, roww, cph)
        acc, den_exp = _merge(out, wc, cph)
        prev_b = b

    sums, mx, cnt = _pool(acc, den_exp, b4.reshape(1, -1), batch_p)
    return _head(sums, mx, cnt, fcW[:128], fcW[128:], fcb.reshape(1, 6))
